# baseline (speedup 1.0000x reference)
import functools

import jax
import jax.numpy as jnp
from jax import lax
from jax.experimental import pallas as pl
from jax.experimental.pallas import tpu as pltpu
from jax.experimental.pallas import tpu_sc as plsc

_L = 16
_NC = 2
_NS = 16
_NW = _NC * _NS
_C = 12800
_NBINS = 128
_NEDGES = 129
_EPAD = 144


def _sc_run(n_per_w, n_chunks):
    assert n_chunks % 2 == 0
    n_pairs = n_chunks // 2
    mesh = plsc.VectorSubcoreMesh(core_axis_name="c", subcore_axis_name="s")
    n_total = n_per_w * _NW

    @functools.partial(
        pl.kernel,
        mesh=mesh,
        compiler_params=pltpu.CompilerParams(
            needs_layout_passes=False, use_tc_tiling_on_sc=False),
        out_type=(
            jax.ShapeDtypeStruct((n_total // 4,), jnp.int32),
            jax.ShapeDtypeStruct((n_total,), jnp.float32),
            jax.ShapeDtypeStruct((_NW, _NBINS), jnp.int32),
        ),
        scratch_types=[
            pltpu.VMEM((_C,), jnp.float32),
            pltpu.VMEM((_C,), jnp.float32),
            pltpu.VMEM((_C // 4,), jnp.int32),
            pltpu.VMEM((_C // 4,), jnp.int32),
            pltpu.VMEM((_C,), jnp.float32),
            pltpu.VMEM((_C,), jnp.float32),
            pltpu.VMEM((_EPAD,), jnp.float32),
            pltpu.VMEM((_NBINS,), jnp.float32),
            pltpu.VMEM((64,), jnp.float32),
            pltpu.VMEM((_L * _NBINS,), jnp.int32),
            pltpu.VMEM((_NBINS,), jnp.int32),
            pltpu.SemaphoreType.DMA,
            pltpu.SemaphoreType.DMA,
            pltpu.SemaphoreType.DMA,
            pltpu.SemaphoreType.DMA,
            pltpu.SemaphoreType.DMA,
            pltpu.SemaphoreType.DMA,
        ],
    )
    def run(x_hbm, edges_hbm, mids_hbm, par_hbm,
            q_hbm, dq_hbm, hist_hbm,
            xin0, xin1, qpk0, qpk1, dqout0, dqout1,
            edges, mids, par, histl, hacc,
            isem0, isem1, qsem0, qsem1, dsem0, dsem1):
        wid = lax.axis_index("s") * _NC + lax.axis_index("c")
        base = wid * n_per_w

        pltpu.sync_copy(edges_hbm, edges)
        pltpu.sync_copy(mids_hbm, mids)
        pltpu.sync_copy(par_hbm, par)

        inv = par[pl.ds(0, _L)]
        c0 = par[pl.ds(_L, _L)]
        stepv = par[pl.ds(2 * _L, _L)]
        m0v = par[pl.ds(3 * _L, _L)]
        lane_off = lax.iota(jnp.int32, _L) * _NBINS
        ones = jnp.ones((_L,), jnp.int32)
        zeros_i = jnp.zeros((_L,), jnp.int32)

        def zero_body(i, _):
            histl[pl.ds(pl.multiple_of(i * _L, _L), _L)] = zeros_i
            return 0
        lax.fori_loop(0, (_L * _NBINS) // _L, zero_body, 0)

        def make_passes(xin, qpk, dqout):
            def _quant(i):
                goff = pl.multiple_of(i * 4 * _L, _L)
                w = None
                for c in range(4):
                    off = goff + c * _L
                    xv = xin[pl.ds(off, _L)]
                    xm = jnp.where(xv != 5.0, xv, 0.0)
                    t = xm * inv + c0
                    k0 = t.astype(jnp.int32)
                    k0 = jnp.minimum(jnp.maximum(k0, 0), _NEDGES - 1)
                    bk = plsc.load_gather(edges, [k0])
                    q = jnp.where(xm <= bk, k0 - 1, k0)
                    q = jnp.minimum(jnp.maximum(q, 0), _NBINS - 1)
                    dq = q.astype(jnp.float32) * stepv + m0v
                    dqout[pl.ds(off, _L)] = dq
                    qc = lax.shift_left(q, jnp.int32(8 * c)) if c else q
                    w = qc if w is None else lax.bitwise_or(w, qc)
                qpk[pl.ds(pl.multiple_of(i * _L, _L), _L)] = w

            def _hist(i, _):
                woff = pl.multiple_of(i * _L, _L)
                wv = qpk[pl.ds(woff, _L)]
                m255 = jnp.int32(255)
                for c in range(4):
                    qc = lax.bitwise_and(
                        lax.shift_right_logical(wv, jnp.int32(8 * c)), m255)
                    plsc.addupdate_scatter(histl, [lane_off + qc], ones)
                return 0
            return _quant, _hist

        quant0, hist0 = make_passes(xin0, qpk0, dqout0)
        quant1, hist1 = make_passes(xin1, qpk1, dqout1)

        def run_main(quant, hist):
            plsc.parallel_loop(0, _C // (4 * _L), unroll=2)(quant)
            lax.fori_loop(0, _C // (4 * _L), hist, 0, unroll=4)

        def wait_in0():
            pltpu.make_async_copy(x_hbm.at[pl.ds(base, _C)], xin0, isem0).wait()

        def wait_in1():
            pltpu.make_async_copy(x_hbm.at[pl.ds(base, _C)], xin1, isem1).wait()

        wbase = pl.multiple_of(base // 4, 8)

        def wait_out0():
            pltpu.make_async_copy(
                qpk0, q_hbm.at[pl.ds(wbase, _C // 4)], qsem0).wait()
            pltpu.make_async_copy(dqout0, dq_hbm.at[pl.ds(base, _C)], dsem0).wait()

        def wait_out1():
            pltpu.make_async_copy(
                qpk1, q_hbm.at[pl.ds(wbase, _C // 4)], qsem1).wait()
            pltpu.make_async_copy(dqout1, dq_hbm.at[pl.ds(base, _C)], dsem1).wait()

        pltpu.async_copy(x_hbm.at[pl.ds(base, _C)], xin0, isem0)

        def pair_body(p, _):
            cb0 = base + (2 * p) * _C
            cb1 = cb0 + _C
            pltpu.async_copy(x_hbm.at[pl.ds(cb1, _C)], xin1, isem1)
            wait_in0()

            @pl.when(p >= 1)
            def _():
                wait_out0()
            run_main(quant0, hist0)
            pltpu.async_copy(
                qpk0, q_hbm.at[pl.ds(pl.multiple_of(cb0 // 4, 8), _C // 4)],
                qsem0)
            pltpu.async_copy(dqout0, dq_hbm.at[pl.ds(cb0, _C)], dsem0)

            @pl.when(p + 1 < n_pairs)
            def _():
                pltpu.async_copy(
                    x_hbm.at[pl.ds(cb0 + 2 * _C, _C)], xin0, isem0)
            wait_in1()

            @pl.when(p >= 1)
            def _():
                wait_out1()
            run_main(quant1, hist1)
            pltpu.async_copy(
                qpk1, q_hbm.at[pl.ds(pl.multiple_of(cb1 // 4, 8), _C // 4)],
                qsem1)
            pltpu.async_copy(dqout1, dq_hbm.at[pl.ds(cb1, _C)], dsem1)
            return 0
        lax.fori_loop(0, n_pairs, pair_body, 0)

        wait_out0()
        wait_out1()

        for j in range(_NBINS // _L):
            acc = histl[pl.ds(j * _L, _L)]
            for lane in range(1, _L):
                acc = acc + histl[pl.ds(lane * _NBINS + j * _L, _L)]
            hacc[pl.ds(j * _L, _L)] = acc
        pltpu.sync_copy(hacc, hist_hbm.at[wid])

    return run


def kernel(x, zscore_quantize_buckets):
    b = zscore_quantize_buckets
    xf = x.reshape(-1)
    n = xf.shape[0]
    assert n % (_NW * _C) == 0
    n_per_w = n // _NW
    n_chunks = n_per_w // _C

    edges = jnp.pad(b, (0, _EPAD - _NEDGES))
    mids = (b[:-1] + b[1:]) * 0.5
    inv = jnp.float32(_NBINS) / (b[_NEDGES - 1] - b[0])
    c0 = -b[0] * inv + jnp.float32(5e-4)
    step = (b[_NEDGES - 1] - b[0]) / jnp.float32(_NBINS)
    m0 = (b[0] + b[1]) * 0.5
    par = jnp.concatenate([jnp.full((_L,), inv, jnp.float32),
                           jnp.full((_L,), c0, jnp.float32),
                           jnp.full((_L,), step, jnp.float32),
                           jnp.full((_L,), m0, jnp.float32)])

    qw, dqf, hpart = _sc_run(n_per_w, n_chunks)(xf, edges, mids, par)
    qb = lax.bitcast_convert_type(qw, jnp.int8)
    qf = qb.reshape(n // 64, _L, 4).transpose(0, 2, 1).reshape(n)
    return (qf.astype(jnp.int32).reshape(x.shape),
            dqf.reshape(x.shape), hpart.sum(axis=0))


# speedup vs baseline: 1.1313x; 1.1313x over previous
import functools

import jax
import jax.numpy as jnp
from jax import lax
from jax.experimental import pallas as pl
from jax.experimental.pallas import tpu as pltpu
from jax.experimental.pallas import tpu_sc as plsc

_L = 16
_NC = 2
_NS = 16
_NW = _NC * _NS
_C = 12800
_NBINS = 128
_NEDGES = 129
_EPAD = 144


def _sc_run(n_per_w, n_chunks):
    assert n_chunks % 2 == 0
    n_pairs = n_chunks // 2
    mesh = plsc.VectorSubcoreMesh(core_axis_name="c", subcore_axis_name="s")
    n_total = n_per_w * _NW

    @functools.partial(
        pl.kernel,
        mesh=mesh,
        compiler_params=pltpu.CompilerParams(
            needs_layout_passes=False, use_tc_tiling_on_sc=False),
        out_type=(
            jax.ShapeDtypeStruct((n_total,), jnp.int32),
            jax.ShapeDtypeStruct((n_total,), jnp.float32),
            jax.ShapeDtypeStruct((_NW, _NBINS), jnp.int32),
        ),
        scratch_types=[
            pltpu.VMEM((_C,), jnp.float32),
            pltpu.VMEM((_C,), jnp.float32),
            pltpu.VMEM((_C,), jnp.int32),
            pltpu.VMEM((_C,), jnp.int32),
            pltpu.VMEM((_C,), jnp.float32),
            pltpu.VMEM((_C,), jnp.float32),
            pltpu.VMEM((_EPAD,), jnp.float32),
            pltpu.VMEM((_NBINS,), jnp.float32),
            pltpu.VMEM((32,), jnp.float32),
            pltpu.VMEM((_L * _NBINS,), jnp.int32),
            pltpu.VMEM((_NBINS,), jnp.int32),
            pltpu.SemaphoreType.DMA,
            pltpu.SemaphoreType.DMA,
            pltpu.SemaphoreType.DMA,
            pltpu.SemaphoreType.DMA,
            pltpu.SemaphoreType.DMA,
            pltpu.SemaphoreType.DMA,
        ],
    )
    def run(x_hbm, edges_hbm, mids_hbm, par_hbm,
            q_hbm, dq_hbm, hist_hbm,
            xin0, xin1, qout0, qout1, dqout0, dqout1,
            edges, mids, par, histl, hacc,
            isem0, isem1, qsem0, qsem1, dsem0, dsem1):
        wid = lax.axis_index("s") * _NC + lax.axis_index("c")
        base = wid * n_per_w

        pltpu.sync_copy(edges_hbm, edges)
        pltpu.sync_copy(mids_hbm, mids)
        pltpu.sync_copy(par_hbm, par)

        inv = par[pl.ds(0, _L)]
        c0 = par[pl.ds(_L, _L)]
        lane_off = lax.iota(jnp.int32, _L) * _NBINS
        ones = jnp.ones((_L,), jnp.int32)
        zeros_i = jnp.zeros((_L,), jnp.int32)

        def zero_body(i, _):
            histl[pl.ds(pl.multiple_of(i * _L, _L), _L)] = zeros_i
            return 0
        lax.fori_loop(0, (_L * _NBINS) // _L, zero_body, 0)

        def make_passes(xin, qout, dqout):
            def _quant(i):
                off = pl.multiple_of(i * _L, _L)
                xv = xin[pl.ds(off, _L)]
                xm = jnp.where(xv != 5.0, xv, 0.0)
                t = xm * inv + c0
                k0 = t.astype(jnp.int32)
                k0 = jnp.minimum(jnp.maximum(k0, 0), _NEDGES - 1)
                bk = plsc.load_gather(edges, [k0])
                q = jnp.where(xm <= bk, k0 - 1, k0)
                q = jnp.minimum(jnp.maximum(q, 0), _NBINS - 1)
                dq = plsc.load_gather(mids, [q])
                qout[pl.ds(off, _L)] = q
                dqout[pl.ds(off, _L)] = dq

            def _hist(i, _):
                off = pl.multiple_of(i * _L, _L)
                qv = qout[pl.ds(off, _L)]
                plsc.addupdate_scatter(histl, [lane_off + qv], ones)
                return 0
            return _quant, _hist

        quant0, hist0 = make_passes(xin0, qout0, dqout0)
        quant1, hist1 = make_passes(xin1, qout1, dqout1)

        def run_main(quant, hist):
            plsc.parallel_loop(0, _C // _L, unroll=4)(quant)
            lax.fori_loop(0, _C // _L, hist, 0, unroll=8)

        def wait_in0():
            pltpu.make_async_copy(x_hbm.at[pl.ds(base, _C)], xin0, isem0).wait()

        def wait_in1():
            pltpu.make_async_copy(x_hbm.at[pl.ds(base, _C)], xin1, isem1).wait()

        def wait_out0():
            pltpu.make_async_copy(qout0, q_hbm.at[pl.ds(base, _C)], qsem0).wait()
            pltpu.make_async_copy(dqout0, dq_hbm.at[pl.ds(base, _C)], dsem0).wait()

        def wait_out1():
            pltpu.make_async_copy(qout1, q_hbm.at[pl.ds(base, _C)], qsem1).wait()
            pltpu.make_async_copy(dqout1, dq_hbm.at[pl.ds(base, _C)], dsem1).wait()

        pltpu.async_copy(x_hbm.at[pl.ds(base, _C)], xin0, isem0)

        def pair_body(p, _):
            cb0 = base + (2 * p) * _C
            cb1 = cb0 + _C
            pltpu.async_copy(x_hbm.at[pl.ds(cb1, _C)], xin1, isem1)
            wait_in0()

            @pl.when(p >= 1)
            def _():
                wait_out0()
            run_main(quant0, hist0)
            pltpu.async_copy(qout0, q_hbm.at[pl.ds(cb0, _C)], qsem0)
            pltpu.async_copy(dqout0, dq_hbm.at[pl.ds(cb0, _C)], dsem0)

            @pl.when(p + 1 < n_pairs)
            def _():
                pltpu.async_copy(
                    x_hbm.at[pl.ds(cb0 + 2 * _C, _C)], xin0, isem0)
            wait_in1()

            @pl.when(p >= 1)
            def _():
                wait_out1()
            run_main(quant1, hist1)
            pltpu.async_copy(qout1, q_hbm.at[pl.ds(cb1, _C)], qsem1)
            pltpu.async_copy(dqout1, dq_hbm.at[pl.ds(cb1, _C)], dsem1)
            return 0
        lax.fori_loop(0, n_pairs, pair_body, 0)

        wait_out0()
        wait_out1()

        for j in range(_NBINS // _L):
            acc = histl[pl.ds(j * _L, _L)]
            for lane in range(1, _L):
                acc = acc + histl[pl.ds(lane * _NBINS + j * _L, _L)]
            hacc[pl.ds(j * _L, _L)] = acc
        pltpu.sync_copy(hacc, hist_hbm.at[wid])

    return run


def kernel(x, zscore_quantize_buckets):
    b = zscore_quantize_buckets
    xf = x.reshape(-1)
    n = xf.shape[0]
    assert n % (_NW * _C) == 0
    n_per_w = n // _NW
    n_chunks = n_per_w // _C

    edges = jnp.pad(b, (0, _EPAD - _NEDGES))
    mids = (b[:-1] + b[1:]) * 0.5
    inv = jnp.float32(_NBINS) / (b[_NEDGES - 1] - b[0])
    c0 = -b[0] * inv + jnp.float32(5e-4)
    par = jnp.concatenate([jnp.full((_L,), inv, jnp.float32),
                           jnp.full((_L,), c0, jnp.float32)])

    qf, dqf, hpart = _sc_run(n_per_w, n_chunks)(xf, edges, mids, par)
    return (qf.reshape(x.shape), dqf.reshape(x.shape), hpart.sum(axis=0))
